# baseline (speedup 1.0000x reference)
import jax
import jax.numpy as jnp
from jax import lax
from jax.experimental import pallas as pl

_N = 16384
_C = 1000
_B = 512


def _body(t_ref, x_ref, o_ref):
    i = pl.program_id(0)

    @pl.when(i == 0)
    def _init():
        o_ref[...] = jnp.zeros((1, 1), jnp.float32)

    x = x_ref[...]
    first = jnp.argmax(x, axis=1).astype(jnp.int32)
    t = t_ref[0, 0, :]
    cnt = jnp.sum((first == t).astype(jnp.float32)).reshape(1, 1)
    o_ref[...] = o_ref[...] + cnt

    @pl.when(i == pl.num_programs(0) - 1)
    def _final():
        o_ref[...] = o_ref[...] * (1.0 / _N)


def kernel(true, logits):
    grid = _N // _B
    t3 = true.reshape(grid, 1, _B).astype(jnp.int32)
    out = pl.pallas_call(
        _body,
        grid=(grid,),
        in_specs=[
            pl.BlockSpec((1, 1, _B), lambda i: (i, 0, 0)),
            pl.BlockSpec((_B, _C), lambda i: (i, 0)),
        ],
        out_specs=pl.BlockSpec((1, 1), lambda i: (0, 0)),
        out_shape=jax.ShapeDtypeStruct((1, 1), jnp.float32),
    )(t3, logits)
    return out[0, 0]


# speedup vs baseline: 1.0264x; 1.0264x over previous
import jax
import jax.numpy as jnp
from jax import lax
from jax.experimental import pallas as pl

_N = 16384
_C = 1000
_B = 512


def _body(t_ref, x_ref, o_ref):
    i = pl.program_id(0)

    @pl.when(i == 0)
    def _init():
        o_ref[...] = jnp.zeros((1, 1), jnp.float32)

    x = x_ref[...]
    m = jnp.max(x, axis=1, keepdims=True)
    col = lax.broadcasted_iota(jnp.int32, (_B, _C), 1)
    first = jnp.min(jnp.where(x == m, col, _C), axis=1)
    t = t_ref[0, 0, :]
    cnt = jnp.sum((first == t).astype(jnp.float32)).reshape(1, 1)
    o_ref[...] = o_ref[...] + cnt

    @pl.when(i == pl.num_programs(0) - 1)
    def _final():
        o_ref[...] = o_ref[...] * (1.0 / _N)


def kernel(true, logits):
    grid = _N // _B
    t3 = true.reshape(grid, 1, _B).astype(jnp.int32)
    out = pl.pallas_call(
        _body,
        grid=(grid,),
        in_specs=[
            pl.BlockSpec((1, 1, _B), lambda i: (i, 0, 0)),
            pl.BlockSpec((_B, _C), lambda i: (i, 0)),
        ],
        out_specs=pl.BlockSpec((1, 1), lambda i: (0, 0)),
        out_shape=jax.ShapeDtypeStruct((1, 1), jnp.float32),
    )(t3, logits)
    return out[0, 0]


# speedup vs baseline: 1.1581x; 1.1282x over previous
import jax
import jax.numpy as jnp
from jax import lax
from jax.experimental import pallas as pl

_N = 16384
_C = 1000
_B = 1024


def _body(t_ref, x_ref, o_ref):
    i = pl.program_id(0)

    @pl.when(i == 0)
    def _init():
        o_ref[...] = jnp.zeros((1, 1), jnp.float32)

    x = x_ref[...]
    m = jnp.max(x, axis=1, keepdims=True)
    col = lax.broadcasted_iota(jnp.int32, (_B, _C), 1)
    first = jnp.min(jnp.where(x == m, col, _C), axis=1)
    t = t_ref[0, 0, :]
    cnt = jnp.sum((first == t).astype(jnp.float32)).reshape(1, 1)
    o_ref[...] = o_ref[...] + cnt

    @pl.when(i == pl.num_programs(0) - 1)
    def _final():
        o_ref[...] = o_ref[...] * (1.0 / _N)


def kernel(true, logits):
    grid = _N // _B
    t3 = true.reshape(grid, 1, _B).astype(jnp.int32)
    out = pl.pallas_call(
        _body,
        grid=(grid,),
        in_specs=[
            pl.BlockSpec((1, 1, _B), lambda i: (i, 0, 0)),
            pl.BlockSpec((_B, _C), lambda i: (i, 0)),
        ],
        out_specs=pl.BlockSpec((1, 1), lambda i: (0, 0)),
        out_shape=jax.ShapeDtypeStruct((1, 1), jnp.float32),
    )(t3, logits)
    return out[0, 0]


# speedup vs baseline: 1.2156x; 1.0497x over previous
import jax
import jax.numpy as jnp
from jax import lax
from jax.experimental import pallas as pl

_N = 16384
_C = 1000
_B = 2048


def _body(t_ref, x_ref, o_ref):
    i = pl.program_id(0)

    @pl.when(i == 0)
    def _init():
        o_ref[...] = jnp.zeros((1, 1), jnp.float32)

    x = x_ref[...]
    m = jnp.max(x, axis=1, keepdims=True)
    col = lax.broadcasted_iota(jnp.int32, (_B, _C), 1)
    first = jnp.min(jnp.where(x == m, col, _C), axis=1)
    t = t_ref[0, 0, :]
    cnt = jnp.sum((first == t).astype(jnp.float32)).reshape(1, 1)
    o_ref[...] = o_ref[...] + cnt

    @pl.when(i == pl.num_programs(0) - 1)
    def _final():
        o_ref[...] = o_ref[...] * (1.0 / _N)


def kernel(true, logits):
    grid = _N // _B
    t3 = true.reshape(grid, 1, _B).astype(jnp.int32)
    out = pl.pallas_call(
        _body,
        grid=(grid,),
        in_specs=[
            pl.BlockSpec((1, 1, _B), lambda i: (i, 0, 0)),
            pl.BlockSpec((_B, _C), lambda i: (i, 0)),
        ],
        out_specs=pl.BlockSpec((1, 1), lambda i: (0, 0)),
        out_shape=jax.ShapeDtypeStruct((1, 1), jnp.float32),
    )(t3, logits)
    return out[0, 0]
